# Initial kernel scaffold; baseline (speedup 1.0000x reference)
import jax
import jax.numpy as jnp
from jax.experimental import pallas as pl


def kernel(x, edge_index, edge_weight, y_one_hot, train_mask, W):
    raise NotImplementedError("write your pallas kernel here")


# speedup vs baseline: 1.2126x; 1.2126x over previous
import functools

import jax
import jax.numpy as jnp
from jax import lax
from jax.experimental import pallas as pl
from jax.experimental.pallas import tpu as pltpu
from jax.experimental.pallas import tpu_sc as plsc

N = 10000
E = 320000
D = 128
C = 10
REG = 1e-05

NC = 2
NS = 16
NW = NC * NS

E2 = E + N
KB = 128
EPT = 10368
NBATCH = EPT // KB
E2_PAD = EPT * NW
E2_ALLOC = E2_PAD + 2 * KB


def _sc_gs_body(xin, row_hbm, norm_hbm, out_hbm,
                rowv, normv, rows,
                sem_i0, sem_i1, sem_i2, sem_g0, sem_g1, sem_g2,
                sem_s0, sem_s1, sem_s2):
    c = lax.axis_index("c")
    s = lax.axis_index("s")
    wid = s * NC + c
    sem_i = [sem_i0, sem_i1, sem_i2]
    sem_g = [sem_g0, sem_g1, sem_g2]
    sem_s = [sem_s0, sem_s1, sem_s2]
    ebase = wid * EPT

    def idx_start(b, q):
        base = ebase + b * KB
        pltpu.async_copy(row_hbm.at[pl.ds(base, KB)], rowv.at[q], sem_i[q])
        pltpu.async_copy(norm_hbm.at[pl.ds(base, KB)], normv.at[q], sem_i[q])

    def idx_wait(q):
        pltpu.make_async_copy(row_hbm.at[pl.ds(0, KB)], rowv.at[q],
                              sem_i[q]).wait()
        pltpu.make_async_copy(norm_hbm.at[pl.ds(0, KB)], normv.at[q],
                              sem_i[q]).wait()

    def gather_start(q):
        pltpu.async_copy(xin.at[rowv.at[q]], rows.at[q], sem_g[q])

    def gather_wait(q):
        pltpu.make_async_copy(xin.at[rowv.at[q]], rows.at[q],
                              sem_g[q]).wait()

    def store_start(b, q):
        base = ebase + b * KB
        pltpu.async_copy(rows.at[q], out_hbm.at[pl.ds(base, KB)], sem_s[q])

    def store_wait(q):
        pltpu.make_async_copy(rows.at[q], out_hbm.at[pl.ds(0, KB)],
                              sem_s[q]).wait()

    def scale(q):
        def sc16(i, _):
            nv16 = normv[q, pl.ds(i * 16, 16)]
            for j in range(16):
                nv = nv16[j]
                for r in range(D // 16):
                    sl = pl.ds(r * 16, 16)
                    rows[q, i * 16 + j, sl] = rows[q, i * 16 + j, sl] * nv
            return 0
        lax.fori_loop(0, KB // 16, sc16, 0)

    idx_start(0, 0)
    idx_start(1, 1)
    idx_wait(0)
    gather_start(0)

    def outer(t, _):
        for k in range(3):
            b = t * 3 + k
            qb = k
            qn = (k + 1) % 3
            qp = (k + 2) % 3
            idx_wait(qn)
            gather_start(qn)
            gather_wait(qb)
            scale(qb)
            if k == 0:
                @pl.when(t > 0)
                def _():
                    store_wait(qp)
            else:
                store_wait(qp)
            store_start(b, qb)
            idx_start(b + 2, qp)
        return 0
    lax.fori_loop(0, NBATCH // 3, outer, 0)

    store_wait(2)
    gather_wait(0)
    idx_wait(1)


@functools.cache
def _build_sc_gs():
    mesh = plsc.VectorSubcoreMesh(core_axis_name="c", subcore_axis_name="s")

    @functools.partial(
        pl.kernel,
        out_type=jax.ShapeDtypeStruct((E2_PAD, D), jnp.float32),
        mesh=mesh,
        compiler_params=pltpu.CompilerParams(needs_layout_passes=False,
                                             use_tc_tiling_on_sc=False),
        scratch_types=[
            pltpu.VMEM((3, KB), jnp.int32),
            pltpu.VMEM((3, KB), jnp.float32),
            pltpu.VMEM((3, KB, D), jnp.float32),
            pltpu.SemaphoreType.DMA,
            pltpu.SemaphoreType.DMA,
            pltpu.SemaphoreType.DMA,
            pltpu.SemaphoreType.DMA,
            pltpu.SemaphoreType.DMA,
            pltpu.SemaphoreType.DMA,
            pltpu.SemaphoreType.DMA,
            pltpu.SemaphoreType.DMA,
            pltpu.SemaphoreType.DMA,
        ],
    )
    def _sc_gs_k(xin, row_hbm, norm_hbm, out_hbm, *scratch):
        _sc_gs_body(xin, row_hbm, norm_hbm, out_hbm, *scratch)

    return _sc_gs_k


def _sc_gather_scale(xin, row_pad, norm_pad):
    return _build_sc_gs()(xin, row_pad, norm_pad)


_BLK = 2000


def _xw_body(x_ref, w_ref, o_ref):
    o_ref[...] = lax.dot_general(x_ref[...], w_ref[...],
                                 (((1,), (1,)), ((), ())),
                                 preferred_element_type=jnp.float32)


def _xw(x, W):
    return pl.pallas_call(
        _xw_body,
        grid=(N // _BLK,),
        in_specs=[pl.BlockSpec((_BLK, D), lambda i: (i, 0)),
                  pl.BlockSpec((D, D), lambda i: (0, 0))],
        out_specs=pl.BlockSpec((_BLK, D), lambda i: (i, 0)),
        out_shape=jax.ShapeDtypeStruct((N, D), jnp.float32),
    )(x, W)


def _sig_body(p_ref, o_ref):
    o_ref[...] = 1.0 / (1.0 + jnp.exp(-p_ref[...]))


def _sigmoid(p):
    return pl.pallas_call(
        _sig_body,
        grid=(N // _BLK,),
        in_specs=[pl.BlockSpec((_BLK, D), lambda i: (i, 0))],
        out_specs=pl.BlockSpec((_BLK, D), lambda i: (i, 0)),
        out_shape=jax.ShapeDtypeStruct((N, D), jnp.float32),
    )(p)


def _pred_body(xg_ref, sol_ref, o_ref):
    o_ref[...] = lax.dot_general(xg_ref[...], sol_ref[...],
                                 (((1,), (0,)), ((), ())),
                                 preferred_element_type=jnp.float32)


def _pred(xg, sol):
    return pl.pallas_call(
        _pred_body,
        grid=(N // _BLK,),
        in_specs=[pl.BlockSpec((_BLK, D), lambda i: (i, 0)),
                  pl.BlockSpec((D, C), lambda i: (0, 0))],
        out_specs=pl.BlockSpec((_BLK, C), lambda i: (i, 0)),
        out_shape=jax.ShapeDtypeStruct((N, C), jnp.float32),
    )(xg, sol)


def kernel(x, edge_index, edge_weight, y_one_hot, train_mask, W):
    row = edge_index[0]
    col = edge_index[1]

    xl = _xw(x, W)

    loop = jnp.arange(N, dtype=jnp.int32)
    row_f = jnp.concatenate([row, loop])
    col_f = jnp.concatenate([col, loop])
    ew_f = jnp.concatenate([edge_weight, jnp.ones((N,), jnp.float32)])
    deg = jnp.zeros((N,), jnp.float32).at[col_f].add(ew_f)
    dis = jnp.where(deg > 0, deg ** -0.5, 0.0)
    norm = dis[row_f] * ew_f * dis[col_f]

    zpad_i = jnp.zeros((E2_ALLOC - E2,), jnp.int32)
    row_pad = jnp.concatenate([row_f, zpad_i])
    norm_pad = jnp.concatenate(
        [norm, jnp.zeros((E2_ALLOC - E2,), jnp.float32)])

    msg1 = _sc_gather_scale(xl, row_pad, norm_pad)[:E2]
    out1 = jnp.zeros((N, D), jnp.float32).at[col_f].add(msg1)
    h = _sigmoid(out1)
    msg2 = _sc_gather_scale(h, row_pad, norm_pad)[:E2]
    xg = jnp.zeros((N, D), jnp.float32).at[col_f].add(msg2)

    tm = train_mask.astype(jnp.float32)
    Y_train = y_one_hot * tm[:, None]
    temp_a = (xg * tm[:, None]).T
    before_inv = temp_a @ xg + REG * jnp.eye(D, dtype=jnp.float32)
    solution = jnp.linalg.inv(before_inv) @ (temp_a @ Y_train)
    return _pred(xg, solution)
